# baseline (speedup 1.0000x reference)
import functools

import jax
import jax.numpy as jnp
from jax import lax
from jax.experimental import pallas as pl
from jax.experimental.pallas import tpu as pltpu
from jax.experimental.pallas import tpu_sc as plsc

_INPUT_DIM = 4096
_NUM_WORKERS = 32
_SPLIT = 1
_NBUF = 2


def kernel(raw_data, emb_value, emb_alpha, factor):
    B, C, D, H, W = raw_data.shape
    n_planes = B * C * D
    planes_per_w = n_planes // _NUM_WORKERS
    rows = H // _SPLIT
    steps = planes_per_w * _SPLIT
    col_chunks = W // 16

    raw_chunks = raw_data.reshape(n_planes * _SPLIT, rows, W)
    fsplat = jnp.full((16,), factor, dtype=jnp.float32)

    mesh = plsc.VectorSubcoreMesh(core_axis_name="c", subcore_axis_name="s")

    @functools.partial(
        pl.kernel,
        mesh=mesh,
        compiler_params=pltpu.CompilerParams(needs_layout_passes=False),
        out_type=jax.ShapeDtypeStruct((B, 4, D, H, W), jnp.float32),
        scratch_types=[
            pltpu.VMEM((_INPUT_DIM,), jnp.float32),
            pltpu.VMEM((_INPUT_DIM,), jnp.float32),
            pltpu.VMEM((16,), jnp.float32),
            pltpu.VMEM((_NBUF, rows, W), jnp.float32),
            pltpu.VMEM((_NBUF, rows, W), jnp.float32),
            pltpu.VMEM((_NBUF, rows, W), jnp.float32),
            pltpu.SemaphoreType.DMA((_NBUF,)),
            pltpu.SemaphoreType.DMA((_NBUF,)),
        ],
    )
    def _lut_kernel(raw_hbm, vtab_hbm, atab_hbm, f_hbm, out_hbm, vtab, atab,
                    fbuf, rawb, vbuf, abuf, in_sem, out_sem):
        wid = lax.axis_index("s") * 2 + lax.axis_index("c")
        base = wid * steps

        def start_in(g, slot):
            return pltpu.async_copy(
                raw_hbm.at[base + g], rawb.at[slot], in_sem.at[slot]
            )

        def start_out(g, slot):
            p = base + g
            b = p // (D * _SPLIT)
            rem = p - b * (D * _SPLIT)
            dpl = rem // _SPLIT
            r0 = (rem - dpl * _SPLIT) * rows
            return [
                pltpu.async_copy(
                    vbuf.at[slot],
                    out_hbm.at[b, c, dpl, pl.ds(r0, rows)],
                    out_sem.at[slot],
                )
                for c in range(3)
            ] + [
                pltpu.async_copy(
                    abuf.at[slot],
                    out_hbm.at[b, 3, dpl, pl.ds(r0, rows)],
                    out_sem.at[slot],
                )
            ]

        in_handles = [None] * _NBUF
        out_handles = [None] * _NBUF
        for g in range(min(_NBUF, steps)):
            in_handles[g] = start_in(g, g)

        pltpu.sync_copy(vtab_hbm, vtab)
        pltpu.sync_copy(atab_hbm, atab)
        pltpu.sync_copy(f_hbm, fbuf)
        fvec = fbuf[...]

        for g in range(steps):
            slot = g % _NBUF
            in_handles[slot].wait()
            if out_handles[slot] is not None:
                for h in out_handles[slot]:
                    h.wait()

            @plsc.parallel_loop(0, rows, 1, unroll=1)
            def inner(r, slot=slot):
                for cc in range(col_chunks):
                    x = rawb[slot, r, pl.ds(cc * 16, 16)]
                    idx = (x * (_INPUT_DIM - 1)).astype(jnp.int32)
                    v = plsc.load_gather(vtab, [idx])
                    a = plsc.load_gather(atab, [idx])
                    vbuf[slot, r, pl.ds(cc * 16, 16)] = jnp.clip(
                        v, 0.0, 1.0
                    )
                    abuf[slot, r, pl.ds(cc * 16, 16)] = (
                        jnp.clip(a, 0.0, 1.0) * fvec
                    )

            out_handles[slot] = start_out(g, slot)
            if g + _NBUF < steps:
                in_handles[slot] = start_in(g + _NBUF, slot)

        for hs in out_handles:
            if hs is not None:
                for h in hs:
                    h.wait()

    return _lut_kernel(
        raw_chunks, emb_value.reshape(-1), emb_alpha.reshape(-1), fsplat
    )


# speedup vs baseline: 1.0200x; 1.0200x over previous
import functools

import jax
import jax.numpy as jnp
from jax import lax
from jax.experimental import pallas as pl
from jax.experimental.pallas import tpu as pltpu
from jax.experimental.pallas import tpu_sc as plsc

_INPUT_DIM = 4096
_NUM_WORKERS = 32
_SPLIT = 1
_NBUF = 2


def kernel(raw_data, emb_value, emb_alpha, factor):
    B, C, D, H, W = raw_data.shape
    n_planes = B * C * D
    planes_per_w = n_planes // _NUM_WORKERS
    rows = H // _SPLIT
    steps = planes_per_w * _SPLIT
    col_chunks = W // 16

    raw_chunks = raw_data.reshape(n_planes * _SPLIT, rows, W)
    vtab_host = jnp.clip(emb_value.reshape(-1), 0.0, 1.0)
    atab_host = jnp.clip(emb_alpha.reshape(-1), 0.0, 1.0) * jnp.asarray(
        factor, jnp.float32
    )

    mesh = plsc.VectorSubcoreMesh(core_axis_name="c", subcore_axis_name="s")

    @functools.partial(
        pl.kernel,
        mesh=mesh,
        compiler_params=pltpu.CompilerParams(needs_layout_passes=False),
        out_type=jax.ShapeDtypeStruct((B, 4, D, H, W), jnp.float32),
        scratch_types=[
            pltpu.VMEM((_INPUT_DIM,), jnp.float32),
            pltpu.VMEM((_INPUT_DIM,), jnp.float32),
            pltpu.VMEM((_NBUF, rows, W), jnp.float32),
            pltpu.VMEM((_NBUF, rows, W), jnp.float32),
            pltpu.VMEM((_NBUF, rows, W), jnp.float32),
            pltpu.SemaphoreType.DMA((_NBUF,)),
            pltpu.SemaphoreType.DMA((_NBUF,)),
        ],
    )
    def _lut_kernel(raw_hbm, vtab_hbm, atab_hbm, out_hbm, vtab, atab,
                    rawb, vbuf, abuf, in_sem, out_sem):
        wid = lax.axis_index("s") * 2 + lax.axis_index("c")
        base = wid * steps

        def start_in(g, slot):
            return pltpu.async_copy(
                raw_hbm.at[base + g], rawb.at[slot], in_sem.at[slot]
            )

        def start_out(g, slot):
            p = base + g
            b = p // (D * _SPLIT)
            rem = p - b * (D * _SPLIT)
            dpl = rem // _SPLIT
            r0 = (rem - dpl * _SPLIT) * rows
            return [
                pltpu.async_copy(
                    vbuf.at[slot],
                    out_hbm.at[b, c, dpl, pl.ds(r0, rows)],
                    out_sem.at[slot],
                )
                for c in range(3)
            ] + [
                pltpu.async_copy(
                    abuf.at[slot],
                    out_hbm.at[b, 3, dpl, pl.ds(r0, rows)],
                    out_sem.at[slot],
                )
            ]

        in_handles = [None] * _NBUF
        out_handles = [None] * _NBUF
        for g in range(min(_NBUF, steps)):
            in_handles[g] = start_in(g, g)

        pltpu.sync_copy(vtab_hbm, vtab)
        pltpu.sync_copy(atab_hbm, atab)

        for g in range(steps):
            slot = g % _NBUF
            in_handles[slot].wait()
            if out_handles[slot] is not None:
                for h in out_handles[slot]:
                    h.wait()

            @plsc.parallel_loop(0, rows, 1, unroll=1)
            def inner(r, slot=slot):
                for cc in range(col_chunks):
                    x = rawb[slot, r, pl.ds(cc * 16, 16)]
                    idx = (x * (_INPUT_DIM - 1)).astype(jnp.int32)
                    vbuf[slot, r, pl.ds(cc * 16, 16)] = plsc.load_gather(
                        vtab, [idx]
                    )
                    abuf[slot, r, pl.ds(cc * 16, 16)] = plsc.load_gather(
                        atab, [idx]
                    )

            out_handles[slot] = start_out(g, slot)
            if g + _NBUF < steps:
                in_handles[slot] = start_in(g + _NBUF, slot)

        for hs in out_handles:
            if hs is not None:
                for h in hs:
                    h.wait()

    return _lut_kernel(raw_chunks, vtab_host, atab_host)
